# baseline (speedup 1.0000x reference)
import functools

import jax
import jax.numpy as jnp
from jax.experimental import pallas as pl
from jax.experimental.pallas import tpu as pltpu

NC = 80
EMB = 128
PAD = 128
NIMG = 2


def _silu(x):
    return x * jax.nn.sigmoid(x)


def _head_kernel(
    x_ref,
    wc0_ref, wc1_ref, wr0_ref, wr1_ref, we_ref,
    pw_ref, bias_ref,
    cls_ref, reg_ref, obj_ref, emb_ref,
    xp0_ref, xp1_ref, xc0_ref, xc1_ref, y10_ref, y11_ref,
    w1_ref, wc2_ref, wr2_ref,
    *, H, W,
):
    C = wc0_ref.shape[1]
    HW = H * W

    @pl.when(pl.program_id(0) == 0)
    def _assemble_weights():
        for t in range(9):
            w1_ref[0:C, t * C : (t + 1) * C] = wc0_ref[t]
            w1_ref[C : 2 * C, t * C : (t + 1) * C] = wr0_ref[t]
            w1_ref[2 * C : 3 * C, t * C : (t + 1) * C] = we_ref[t]
            wc2_ref[:, t * C : (t + 1) * C] = wc1_ref[t]
            wr2_ref[:, t * C : (t + 1) * C] = wr1_ref[t]

    b1 = bias_ref[0 : 3 * C]
    bc2 = bias_ref[3 * C : 4 * C]
    br2 = bias_ref[4 * C : 5 * C]
    cpb = bias_ref[5 * C : 5 * C + NC]
    rpob = bias_ref[5 * C + NC : 5 * C + NC + 5]
    epb = bias_ref[5 * C + NC + 5 : 5 * C + NC + 5 + EMB]

    lane = jax.lax.broadcasted_iota(jnp.int32, (1, HW), 1)
    wcol = lane & (W - 1)
    mask_m = (wcol != 0)
    mask_p = (wcol != W - 1)

    def build_xc(xp_ref, xc_ref):
        for ky in range(3):
            for kx in range(3):
                o = (ky - 1) * W + (kx - 1)
                s = xp_ref[:, PAD + o : PAD + o + HW]
                if kx == 0:
                    s = jnp.where(mask_m, s, jnp.bfloat16(0))
                elif kx == 2:
                    s = jnp.where(mask_p, s, jnp.bfloat16(0))
                idx = 3 * ky + kx
                xc_ref[idx * C : (idx + 1) * C, :] = s

    def mm(a, b):
        return jnp.dot(a, b, preferred_element_type=jnp.float32)

    for xp_ref in (xp0_ref, xp1_ref):
        xp_ref[:, 0:PAD] = jnp.zeros((C, PAD), jnp.bfloat16)
        xp_ref[:, PAD + HW : 2 * PAD + HW] = jnp.zeros((C, PAD), jnp.bfloat16)

    for img in range(NIMG):
        xp_ref = (xp0_ref, xp1_ref)[img]
        xc_ref = (xc0_ref, xc1_ref)[img]
        y1_ref = (y10_ref, y11_ref)[img]

        xp_ref[:, PAD : PAD + HW] = x_ref[img].astype(jnp.bfloat16)
        build_xc(xp_ref, xc_ref)
        y1_ref[...] = _silu(mm(w1_ref[...], xc_ref[...]) + b1).astype(
            jnp.bfloat16
        )

        xp_ref[:, PAD : PAD + HW] = y1_ref[0:C, :]
        build_xc(xp_ref, xc_ref)
        c2 = _silu(mm(wc2_ref[...], xc_ref[...]) + bc2).astype(jnp.bfloat16)
        cls_ref[img] = mm(pw_ref[0:NC], c2) + cpb

        xp_ref[:, PAD : PAD + HW] = y1_ref[C : 2 * C, :]
        build_xc(xp_ref, xc_ref)
        r2 = _silu(mm(wr2_ref[...], xc_ref[...]) + br2).astype(jnp.bfloat16)
        t = mm(pw_ref[NC : NC + 5], r2) + rpob
        reg_ref[img] = t[0:4, :]
        obj_ref[img] = t[4:5, :]

        e = mm(pw_ref[NC + 5 : NC + 5 + EMB], y1_ref[2 * C : 3 * C, :]) + epb
        n = jnp.sqrt(jnp.sum(e * e, axis=0, keepdims=True))
        emb_ref[img] = e / jnp.maximum(n, 1e-12)


def _scale_head(feat, cw0, cb0, cw1, cb1, rw0, rb0, rw1, rb1,
                cpw, cpb, rpw, rpb, opw, opb, ew, eb, epw, epb):
    Bn, C, H, W = feat.shape
    HW = H * W
    xf = feat.reshape(Bn, C, HW)

    def w9(w):
        return (
            w.astype(jnp.bfloat16).transpose(2, 3, 0, 1).reshape(9, C, C)
        )

    pw = jnp.concatenate(
        [cpw[:, :, 0, 0], rpw[:, :, 0, 0], opw[:, :, 0, 0], epw[:, :, 0, 0]],
        axis=0,
    ).astype(jnp.bfloat16)
    bias = jnp.concatenate(
        [cb0, rb0, eb, cb1, rb1, cpb, rpb, opb, epb]
    ).reshape(-1, 1)

    args = (xf, w9(cw0), w9(cw1), w9(rw0), w9(rw1), w9(ew), pw, bias)

    in_specs = [
        pl.BlockSpec((NIMG, C, HW), lambda b: (b, 0, 0))
    ] + [
        pl.BlockSpec(a.shape, lambda b, _n=len(a.shape): (0,) * _n)
        for a in args[1:]
    ]

    out_shapes = [
        jax.ShapeDtypeStruct((Bn, NC, HW), jnp.float32),
        jax.ShapeDtypeStruct((Bn, 4, HW), jnp.float32),
        jax.ShapeDtypeStruct((Bn, 1, HW), jnp.float32),
        jax.ShapeDtypeStruct((Bn, EMB, HW), jnp.float32),
    ]
    out_specs = [
        pl.BlockSpec((NIMG, s.shape[1], HW), lambda b: (b, 0, 0))
        for s in out_shapes
    ]

    cls, reg, obj, emb = pl.pallas_call(
        functools.partial(_head_kernel, H=H, W=W),
        grid=(Bn // NIMG,),
        in_specs=in_specs,
        out_specs=out_specs,
        out_shape=out_shapes,
        scratch_shapes=[
            pltpu.VMEM((C, HW + 2 * PAD), jnp.bfloat16),
            pltpu.VMEM((C, HW + 2 * PAD), jnp.bfloat16),
            pltpu.VMEM((9 * C, HW), jnp.bfloat16),
            pltpu.VMEM((9 * C, HW), jnp.bfloat16),
            pltpu.VMEM((3 * C, HW), jnp.bfloat16),
            pltpu.VMEM((3 * C, HW), jnp.bfloat16),
            pltpu.VMEM((3 * C, 9 * C), jnp.bfloat16),
            pltpu.VMEM((C, 9 * C), jnp.bfloat16),
            pltpu.VMEM((C, 9 * C), jnp.bfloat16),
        ],
    )(*args)

    def shape4(y):
        return y.reshape(Bn, -1, H, W)

    return shape4(cls), shape4(reg), shape4(obj), shape4(emb)


def kernel(feat0, feat1, feat2,
           cls_w_0_0, cls_b_0_0, cls_w_0_1, cls_b_0_1,
           reg_w_0_0, reg_b_0_0, reg_w_0_1, reg_b_0_1,
           cls_pw_0, cls_pb_0, reg_pw_0, reg_pb_0, obj_pw_0, obj_pb_0,
           emb_w_0, emb_b_0, emb_pw_0, emb_pb_0,
           cls_w_1_0, cls_b_1_0, cls_w_1_1, cls_b_1_1,
           reg_w_1_0, reg_b_1_0, reg_w_1_1, reg_b_1_1,
           cls_pw_1, cls_pb_1, reg_pw_1, reg_pb_1, obj_pw_1, obj_pb_1,
           emb_w_1, emb_b_1, emb_pw_1, emb_pb_1,
           cls_w_2_0, cls_b_2_0, cls_w_2_1, cls_b_2_1,
           reg_w_2_0, reg_b_2_0, reg_w_2_1, reg_b_2_1,
           cls_pw_2, cls_pb_2, reg_pw_2, reg_pb_2, obj_pw_2, obj_pb_2,
           emb_w_2, emb_b_2, emb_pw_2, emb_pb_2):
    feats = [feat0, feat1, feat2]
    p = dict(locals())
    cls_outs, reg_outs, obj_outs, emb_outs = [], [], [], []
    for i, feat in enumerate(feats):
        c, r, o, e = _scale_head(
            feat,
            p[f'cls_w_{i}_0'], p[f'cls_b_{i}_0'],
            p[f'cls_w_{i}_1'], p[f'cls_b_{i}_1'],
            p[f'reg_w_{i}_0'], p[f'reg_b_{i}_0'],
            p[f'reg_w_{i}_1'], p[f'reg_b_{i}_1'],
            p[f'cls_pw_{i}'], p[f'cls_pb_{i}'],
            p[f'reg_pw_{i}'], p[f'reg_pb_{i}'],
            p[f'obj_pw_{i}'], p[f'obj_pb_{i}'],
            p[f'emb_w_{i}'], p[f'emb_b_{i}'],
            p[f'emb_pw_{i}'], p[f'emb_pb_{i}'],
        )
        cls_outs.append(c)
        reg_outs.append(r)
        obj_outs.append(o)
        emb_outs.append(e)
    return tuple(cls_outs + reg_outs + obj_outs + emb_outs)


# speedup vs baseline: 1.0795x; 1.0795x over previous
import functools

import jax
import jax.numpy as jnp
from jax.experimental import pallas as pl
from jax.experimental.pallas import tpu as pltpu

NC = 80
EMB = 128
PAD = 128
NIMG = 2


def _silu(x):
    return x * jax.nn.sigmoid(x)


def _head_kernel(
    x_ref,
    wc0_ref, wc1_ref, wr0_ref, wr1_ref, we_ref,
    pw_ref, bias_ref,
    cls_ref, reg_ref, obj_ref, emb_ref,
    xp0_ref, xp1_ref, xc0_ref, xc1_ref, y10_ref, y11_ref,
    w1_ref, wc2_ref, wr2_ref,
    *, H, W,
):
    C = wc0_ref.shape[1]
    HW = H * W

    @pl.when(pl.program_id(0) == 0)
    def _assemble_weights():
        for t in range(9):
            w1_ref[0:C, t * C : (t + 1) * C] = wc0_ref[t]
            w1_ref[C : 2 * C, t * C : (t + 1) * C] = wr0_ref[t]
            w1_ref[2 * C : 3 * C, t * C : (t + 1) * C] = we_ref[t]
            wc2_ref[:, t * C : (t + 1) * C] = wc1_ref[t]
            wr2_ref[:, t * C : (t + 1) * C] = wr1_ref[t]

    b1 = bias_ref[0 : 3 * C]
    bc2 = bias_ref[3 * C : 4 * C]
    br2 = bias_ref[4 * C : 5 * C]
    cpb = bias_ref[5 * C : 5 * C + NC]
    rpob = bias_ref[5 * C + NC : 5 * C + NC + 5]
    epb = bias_ref[5 * C + NC + 5 : 5 * C + NC + 5 + EMB]

    lane = jax.lax.broadcasted_iota(jnp.int32, (1, HW), 1)
    wcol = lane & (W - 1)
    mask_m = (wcol != 0)
    mask_p = (wcol != W - 1)

    def build_xc(xp_ref, xc_ref):
        for ky in range(3):
            for kx in range(3):
                o = (ky - 1) * W + (kx - 1)
                s = xp_ref[:, PAD + o : PAD + o + HW]
                if kx == 0:
                    s = jnp.where(mask_m, s, jnp.bfloat16(0))
                elif kx == 2:
                    s = jnp.where(mask_p, s, jnp.bfloat16(0))
                idx = 3 * ky + kx
                xc_ref[idx * C : (idx + 1) * C, :] = s

    def mm(a, b):
        return jnp.dot(a, b, preferred_element_type=jnp.float32)

    for xp_ref in (xp0_ref, xp1_ref):
        xp_ref[:, 0:PAD] = jnp.zeros((C, PAD), jnp.bfloat16)
        xp_ref[:, PAD + HW : 2 * PAD + HW] = jnp.zeros((C, PAD), jnp.bfloat16)

    for img in range(NIMG):
        xp_ref = (xp0_ref, xp1_ref)[img]
        xc_ref = (xc0_ref, xc1_ref)[img]
        y1_ref = (y10_ref, y11_ref)[img]

        xp_ref[:, PAD : PAD + HW] = x_ref[img].astype(jnp.bfloat16)
        build_xc(xp_ref, xc_ref)
        y1_ref[...] = _silu(mm(w1_ref[...], xc_ref[...]) + b1).astype(
            jnp.bfloat16
        )

        xp_ref[:, PAD : PAD + HW] = y1_ref[0:C, :]
        build_xc(xp_ref, xc_ref)
        c2 = _silu(mm(wc2_ref[...], xc_ref[...]) + bc2).astype(jnp.bfloat16)
        cls_ref[img] = mm(pw_ref[0:NC], c2) + cpb

        xp_ref[:, PAD : PAD + HW] = y1_ref[C : 2 * C, :]
        build_xc(xp_ref, xc_ref)
        r2 = _silu(mm(wr2_ref[...], xc_ref[...]) + br2).astype(jnp.bfloat16)
        t = mm(pw_ref[NC : NC + 5], r2) + rpob
        reg_ref[img] = t[0:4, :]
        obj_ref[img] = t[4:5, :]

        e = mm(pw_ref[NC + 5 : NC + 5 + EMB], y1_ref[2 * C : 3 * C, :]) + epb
        n = jnp.sqrt(jnp.sum(e * e, axis=0, keepdims=True))
        emb_ref[img] = e / jnp.maximum(n, 1e-12)


def _scale_head(feat, cw0, cb0, cw1, cb1, rw0, rb0, rw1, rb1,
                cpw, cpb, rpw, rpb, opw, opb, ew, eb, epw, epb):
    Bn, C, H, W = feat.shape
    HW = H * W
    xf = feat.reshape(Bn, C, HW)

    def w9(w):
        return (
            w.astype(jnp.bfloat16).transpose(0, 3, 4, 1, 2).reshape(45, C, C)
        )

    pw = jnp.concatenate(
        [cpw[:, :, 0, 0], rpw[:, :, 0, 0], opw[:, :, 0, 0], epw[:, :, 0, 0]],
        axis=0,
    ).astype(jnp.bfloat16)
    bias = jnp.concatenate(
        [cb0, rb0, eb, cb1, rb1, cpb, rpb, opb, epb]
    ).reshape(-1, 1)

    wall = w9(jnp.stack([cw0, cw1, rw0, rw1, ew]))
    args = (xf, wall[0:9], wall[9:18], wall[18:27], wall[27:36],
            wall[36:45], pw, bias)

    in_specs = [
        pl.BlockSpec((NIMG, C, HW), lambda b: (b, 0, 0))
    ] + [
        pl.BlockSpec(a.shape, lambda b, _n=len(a.shape): (0,) * _n)
        for a in args[1:]
    ]

    out_shapes = [
        jax.ShapeDtypeStruct((Bn, NC, HW), jnp.float32),
        jax.ShapeDtypeStruct((Bn, 4, HW), jnp.float32),
        jax.ShapeDtypeStruct((Bn, 1, HW), jnp.float32),
        jax.ShapeDtypeStruct((Bn, EMB, HW), jnp.float32),
    ]
    out_specs = [
        pl.BlockSpec((NIMG, s.shape[1], HW), lambda b: (b, 0, 0))
        for s in out_shapes
    ]

    cls, reg, obj, emb = pl.pallas_call(
        functools.partial(_head_kernel, H=H, W=W),
        grid=(Bn // NIMG,),
        in_specs=in_specs,
        out_specs=out_specs,
        out_shape=out_shapes,
        scratch_shapes=[
            pltpu.VMEM((C, HW + 2 * PAD), jnp.bfloat16),
            pltpu.VMEM((C, HW + 2 * PAD), jnp.bfloat16),
            pltpu.VMEM((9 * C, HW), jnp.bfloat16),
            pltpu.VMEM((9 * C, HW), jnp.bfloat16),
            pltpu.VMEM((3 * C, HW), jnp.bfloat16),
            pltpu.VMEM((3 * C, HW), jnp.bfloat16),
            pltpu.VMEM((3 * C, 9 * C), jnp.bfloat16),
            pltpu.VMEM((C, 9 * C), jnp.bfloat16),
            pltpu.VMEM((C, 9 * C), jnp.bfloat16),
        ],
    )(*args)

    def shape4(y):
        return y.reshape(Bn, -1, H, W)

    return shape4(cls), shape4(reg), shape4(obj), shape4(emb)


def kernel(feat0, feat1, feat2,
           cls_w_0_0, cls_b_0_0, cls_w_0_1, cls_b_0_1,
           reg_w_0_0, reg_b_0_0, reg_w_0_1, reg_b_0_1,
           cls_pw_0, cls_pb_0, reg_pw_0, reg_pb_0, obj_pw_0, obj_pb_0,
           emb_w_0, emb_b_0, emb_pw_0, emb_pb_0,
           cls_w_1_0, cls_b_1_0, cls_w_1_1, cls_b_1_1,
           reg_w_1_0, reg_b_1_0, reg_w_1_1, reg_b_1_1,
           cls_pw_1, cls_pb_1, reg_pw_1, reg_pb_1, obj_pw_1, obj_pb_1,
           emb_w_1, emb_b_1, emb_pw_1, emb_pb_1,
           cls_w_2_0, cls_b_2_0, cls_w_2_1, cls_b_2_1,
           reg_w_2_0, reg_b_2_0, reg_w_2_1, reg_b_2_1,
           cls_pw_2, cls_pb_2, reg_pw_2, reg_pb_2, obj_pw_2, obj_pb_2,
           emb_w_2, emb_b_2, emb_pw_2, emb_pb_2):
    feats = [feat0, feat1, feat2]
    p = dict(locals())
    cls_outs, reg_outs, obj_outs, emb_outs = [], [], [], []
    for i, feat in enumerate(feats):
        c, r, o, e = _scale_head(
            feat,
            p[f'cls_w_{i}_0'], p[f'cls_b_{i}_0'],
            p[f'cls_w_{i}_1'], p[f'cls_b_{i}_1'],
            p[f'reg_w_{i}_0'], p[f'reg_b_{i}_0'],
            p[f'reg_w_{i}_1'], p[f'reg_b_{i}_1'],
            p[f'cls_pw_{i}'], p[f'cls_pb_{i}'],
            p[f'reg_pw_{i}'], p[f'reg_pb_{i}'],
            p[f'obj_pw_{i}'], p[f'obj_pb_{i}'],
            p[f'emb_w_{i}'], p[f'emb_b_{i}'],
            p[f'emb_pw_{i}'], p[f'emb_pb_{i}'],
        )
        cls_outs.append(c)
        reg_outs.append(r)
        obj_outs.append(o)
        emb_outs.append(e)
    return tuple(cls_outs + reg_outs + obj_outs + emb_outs)
